# baseline (speedup 1.0000x reference)
import jax
import jax.numpy as jnp
from jax import lax
from jax.experimental import pallas as pl

_H = 256


def _body(oh, ov, ker, imgb, out):
    lint = jax.lax.broadcasted_iota(jnp.int32, (1, _H), 1)
    jlane = lint.astype(jnp.float32)

    li = jax.lax.broadcasted_iota(jnp.int32, (_H, _H), 0)
    co = jax.lax.broadcasted_iota(jnp.int32, (_H, _H), 1)
    sel = (li == 2 * (co % 128) + co // 128).astype(jnp.float32)

    nr, nc = imgb.shape[2], imgb.shape[3]
    r_io = jax.lax.broadcasted_iota(jnp.int32, (nr, nc), 0)
    l_io = jax.lax.broadcasted_iota(jnp.int32, (nr, nc), 1)
    diag = {}
    for s in range(-3, 4):
        mask = (l_io - r_io) == s
        for cch in range(3):
            M = imgb[0, cch]
            bd = jnp.sum(jnp.where(mask, M, 0.0), axis=0, keepdims=True)
            for a in range(4):
                b2 = a + s
                if 0 <= b2 <= 3:
                    diag[(a, b2, cch)] = bd[:, b2:b2 + _H]

    selb = sel.astype(jnp.bfloat16)
    dn = (((1,), (0,)), ((), ()))

    def deint(x):
        h1 = x.astype(jnp.bfloat16)
        r1 = x - h1.astype(jnp.float32)
        h2 = r1.astype(jnp.bfloat16)
        h3 = (r1 - h2.astype(jnp.float32)).astype(jnp.bfloat16)
        d = lax.dot_general(h1, selb, dn, preferred_element_type=jnp.float32)
        d = d + lax.dot_general(h2, selb, dn, preferred_element_type=jnp.float32)
        return d + lax.dot_general(h3, selb, dn, preferred_element_type=jnp.float32)

    bxl, byl, dh, dv = [], [], [], []
    for k in range(9):
        rk, ck = k // 3, k % 3
        xs = ((oh[0, k] + 1.5) + rk) + (jlane + 0.5)
        ys = ((ov[0, k] + 1.5) + ck) + (jlane + 0.5)
        flx, fly = jnp.floor(xs), jnp.floor(ys)
        bxl.append(flx - (jlane + (rk + 2)))
        byl.append(fly - (jlane + (ck + 2)))
        dh.append(deint(xs - flx))
        dv.append(deint(ys - fly))

    def srcw(dlist, k, t):
        q = 2 * k + t
        c, kp = q // 9, q % 9
        dc = dlist[kp][:, 128 * c:128 * c + 128]
        r3 = dc.reshape(128, 2, 128)
        src = jnp.concatenate([r3[:, 0, :], r3[:, 1, :]], axis=1)
        return jnp.concatenate([1.0 - src, src], axis=0)

    acc0 = acc1 = acc2 = None
    for k in range(9):
        rk, ck = k // 3, k % 3
        bx, by = bxl[k], byl[k]
        w0 = srcw(dh, k, 0)
        w1 = srcw(dh, k, 1)
        v0 = srcw(dv, k, 0)
        v1 = srcw(dv, k, 1)
        g = []
        for cch in range(3):
            v00 = diag[(rk, ck, cch)]
            v01 = diag[(rk, ck + 1, cch)]
            v10 = diag[(rk + 1, ck, cch)]
            v11 = diag[(rk + 1, ck + 1, cch)]
            g.append((1 - bx) * ((1 - by) * v00 + by * v01)
                     + bx * ((1 - by) * v10 + by * v11))
        g0, g1, g2 = g
        kv = ker[0, k]
        r0 = v0 * (w0 * g0 + w1 * g0) + v1 * (w0 * g1 + w1 * g2)
        r1 = v0 * (w0 * g0 + w1 * g1) + v1 * (w0 * g1 + w1 * g2)
        r2 = v0 * (w0 * g0 + w1 * g1) + v1 * (w0 * g2 + w1 * g2)
        if acc0 is None:
            acc0, acc1, acc2 = kv * r0, kv * r1, kv * r2
        else:
            acc0, acc1, acc2 = acc0 + kv * r0, acc1 + kv * r1, acc2 + kv * r2

    for cch, acc in enumerate((acc0, acc1, acc2)):
        o = acc * 255.0
        out[0, cch] = o - jnp.sin(2 * jnp.pi * o) / (2 * jnp.pi)


def kernel(img, kernels, offsets_h, offsets_v):
    B = img.shape[0]
    imgb = img[:, :, 2:262, 2:266]

    full = pl.BlockSpec((1, 9, _H, _H), lambda b: (b, 0, 0, 0))
    out = pl.pallas_call(
        _body,
        grid=(B,),
        in_specs=[full, full, full,
                  pl.BlockSpec((1, 3, 260, 264), lambda b: (b, 0, 0, 0))],
        out_specs=pl.BlockSpec((1, 3, _H, _H), lambda b: (b, 0, 0, 0)),
        out_shape=jax.ShapeDtypeStruct((B, 3, _H, _H), jnp.float32),
    )(offsets_h, offsets_v, kernels, imgb)
    return jnp.transpose(out, (0, 2, 3, 1))
